# Initial kernel scaffold; baseline (speedup 1.0000x reference)
import jax
import jax.numpy as jnp
from jax.experimental import pallas as pl


def kernel(x, edge_index, W1, b1, g1, be1, W2, b2, g2, be2, W3, b3, g3, be3, Wc, bc):
    raise NotImplementedError("write your pallas kernel here")


# speedup vs baseline: 4.1820x; 4.1820x over previous
import functools

import jax
import jax.numpy as jnp
from jax import lax
from jax.experimental import pallas as pl
from jax.experimental.pallas import tpu as pltpu
from jax.experimental.pallas import tpu_sc as plsc

N = 10000
D = 128
E = 320000
NC = 2
NS = 16
NW = NC * NS
L = 16
EPT = E // NW
FPT = D // NW
PB = 3200
NP = E // PB

_MESH = plsc.VectorSubcoreMesh(
    core_axis_name="c", subcore_axis_name="s", num_cores=NC, num_subcores=NS)


def _deg_body(dst_hbm, out_hbm, didx, deg):
    c = lax.axis_index("c")
    s = lax.axis_index("s")
    wid = c * NS + s

    zeros = jnp.zeros((L,), jnp.float32)

    def zrow(i, _):
        deg[pl.ds(i * L, L)] = zeros
        return 0
    lax.fori_loop(0, N // L, zrow, 0)

    pltpu.sync_copy(dst_hbm.at[pl.ds(wid * EPT, EPT)], didx)

    ones = jnp.ones((L,), jnp.float32)

    def step(v, _):
        d16 = didx[pl.ds(v * L, L)]
        plsc.addupdate_scatter(deg, [d16], ones)
        return 0
    lax.fori_loop(0, EPT // L, step, 0)

    pltpu.sync_copy(deg, out_hbm.at[wid])


_deg_call = functools.partial(
    pl.kernel,
    out_type=jax.ShapeDtypeStruct((NW, N), jnp.float32),
    mesh=_MESH,
    compiler_params=pltpu.CompilerParams(needs_layout_passes=False),
    scratch_types=[
        pltpu.VMEM((EPT,), jnp.int32),
        pltpu.VMEM((N,), jnp.float32),
    ],
)(_deg_body)


def _agg_body(hpre_hbm, src_hbm, dst_hbm, out_hbm, sidx, didx, hloc, aloc):
    c = lax.axis_index("c")
    s = lax.axis_index("s")
    wid = c * NS + s

    zeros = jnp.zeros((L,), jnp.float32)

    def zrow(i, _):
        aloc[pl.ds(i * L, L)] = zeros
        return 0
    lax.fori_loop(0, (N * FPT) // L, zrow, 0)

    pltpu.sync_copy(hpre_hbm.at[wid], hloc)

    def piece(p, _):
        pltpu.sync_copy(src_hbm.at[pl.ds(p * PB, PB)], sidx)
        pltpu.sync_copy(dst_hbm.at[pl.ds(p * PB, PB)], didx)

        def step(v, _2):
            s16 = sidx[pl.ds(v * L, L)] * FPT
            d16 = didx[pl.ds(v * L, L)] * FPT
            for col in range(FPT):
                g = plsc.load_gather(hloc, [s16 + col])
                plsc.addupdate_scatter(aloc, [d16 + col], g)
            return 0
        lax.fori_loop(0, PB // L, step, 0)
        return 0
    lax.fori_loop(0, NP, piece, 0)

    pltpu.sync_copy(aloc, out_hbm.at[wid])


_agg_call = functools.partial(
    pl.kernel,
    out_type=jax.ShapeDtypeStruct((NW, N * FPT), jnp.float32),
    mesh=_MESH,
    compiler_params=pltpu.CompilerParams(needs_layout_passes=False),
    scratch_types=[
        pltpu.VMEM((PB,), jnp.int32),
        pltpu.VMEM((PB,), jnp.int32),
        pltpu.VMEM((N * FPT,), jnp.float32),
        pltpu.VMEM((N * FPT,), jnp.float32),
    ],
)(_agg_body)


def _to_slices(h):
    return h.reshape(N, NW, FPT).transpose(1, 0, 2).reshape(NW, N * FPT)


def _from_slices(a):
    return a.reshape(NW, N, FPT).transpose(1, 0, 2).reshape(N, D)


def _pre_body(degp_ref, x_ref, w_ref, dinv_ref, hpre_ref):
    deg = jnp.sum(degp_ref[...], axis=1, keepdims=True) + 1.0
    dinv = lax.rsqrt(jnp.maximum(deg, 1e-12))
    h = jnp.dot(x_ref[...], w_ref[...], preferred_element_type=jnp.float32)
    dinv_ref[...] = dinv
    hpre_ref[...] = h * dinv


def _pre_call(degp, x, w):
    return pl.pallas_call(
        _pre_body,
        out_shape=(
            jax.ShapeDtypeStruct((N, 1), jnp.float32),
            jax.ShapeDtypeStruct((N, D), jnp.float32),
        ),
    )(degp, x, w)


def _bn(t, g, be):
    mu = jnp.mean(t, axis=0, keepdims=True)
    var = jnp.mean((t - mu) * (t - mu), axis=0, keepdims=True)
    return (t - mu) * lax.rsqrt(var + 1e-5) * g + be


def _mid_body(agg_ref, hprev_ref, dinv_ref, g_ref, be_ref, w_ref, hnext_ref):
    dinv = dinv_ref[...]
    t = (agg_ref[...] + hprev_ref[...]) * dinv
    z = jnp.maximum(_bn(t, g_ref[...], be_ref[...]), 0.0)
    h = jnp.dot(z, w_ref[...], preferred_element_type=jnp.float32)
    hnext_ref[...] = h * dinv


def _mid_call(agg, hprev, dinv, g, be, w):
    return pl.pallas_call(
        _mid_body,
        out_shape=jax.ShapeDtypeStruct((N, D), jnp.float32),
    )(agg, hprev, dinv, g.reshape(1, D), be.reshape(1, D), w)


def _out_body(agg_ref, hprev_ref, dinv_ref, g_ref, be_ref, wc_ref, bc_ref,
              out_ref, emb_ref):
    t = (agg_ref[...] + hprev_ref[...]) * dinv_ref[...]
    emb = _bn(t, g_ref[...], be_ref[...])
    emb_ref[...] = emb
    out_ref[...] = jnp.dot(emb, wc_ref[...],
                           preferred_element_type=jnp.float32) + bc_ref[...]


def _out_call(agg, hprev, dinv, g, be, wc, bc):
    nout = wc.shape[1]
    return pl.pallas_call(
        _out_body,
        out_shape=(
            jax.ShapeDtypeStruct((N, nout), jnp.float32),
            jax.ShapeDtypeStruct((N, D), jnp.float32),
        ),
    )(agg, hprev, dinv, g.reshape(1, D), be.reshape(1, D), wc,
      bc.reshape(1, nout))


def kernel(x, edge_index, W1, b1, g1, be1, W2, b2, g2, be2, W3, b3, g3, be3,
           Wc, bc):
    src = edge_index[0]
    dst = edge_index[1]

    degp = _deg_call(dst)
    dinv, hpre1 = _pre_call(degp.T, x, W1)

    agg1 = _from_slices(_agg_call(_to_slices(hpre1), src, dst))
    hpre2 = _mid_call(agg1, hpre1, dinv, g1, be1, W2)

    agg2 = _from_slices(_agg_call(_to_slices(hpre2), src, dst))
    hpre3 = _mid_call(agg2, hpre2, dinv, g2, be2, W3)

    agg3 = _from_slices(_agg_call(_to_slices(hpre3), src, dst))
    out, emb = _out_call(agg3, hpre3, dinv, g3, be3, Wc, bc)
    return (out, emb)


# speedup vs baseline: 7.2734x; 1.7392x over previous
import functools

import jax
import jax.numpy as jnp
from jax import lax
from jax.experimental import pallas as pl
from jax.experimental.pallas import tpu as pltpu
from jax.experimental.pallas import tpu_sc as plsc

N = 10000
D = 128
E = 320000
NC = 2
NS = 16
NW = NC * NS
L = 16
EPT = E // NW
FPT = D // NW
PB = 3200
NP = E // PB

_MESH = plsc.VectorSubcoreMesh(
    core_axis_name="c", subcore_axis_name="s", num_cores=NC, num_subcores=NS)


def _deg_body(dst_hbm, out_hbm, didx, deg):
    c = lax.axis_index("c")
    s = lax.axis_index("s")
    wid = c * NS + s

    zeros = jnp.zeros((L,), jnp.float32)

    def zrow(i, _):
        deg[pl.ds(i * L, L)] = zeros
        return 0
    lax.fori_loop(0, N // L, zrow, 0)

    pltpu.sync_copy(dst_hbm.at[pl.ds(wid * EPT, EPT)], didx)

    ones = jnp.ones((L,), jnp.float32)

    def step(v, _):
        d16 = didx[pl.ds(v * L, L)]
        plsc.addupdate_scatter(deg, [d16], ones)
        return 0
    lax.fori_loop(0, EPT // L, step, 0)

    pltpu.sync_copy(deg, out_hbm.at[wid])


_deg_call = functools.partial(
    pl.kernel,
    out_type=jax.ShapeDtypeStruct((NW, N), jnp.float32),
    mesh=_MESH,
    compiler_params=pltpu.CompilerParams(needs_layout_passes=False),
    scratch_types=[
        pltpu.VMEM((EPT,), jnp.int32),
        pltpu.VMEM((N,), jnp.float32),
    ],
)(_deg_body)


def _agg_body(hpre_hbm, src_hbm, dst_hbm, out_hbm, sidx, didx, hloc, aloc):
    c = lax.axis_index("c")
    s = lax.axis_index("s")
    wid = c * NS + s

    zeros = jnp.zeros((L,), jnp.float32)

    def zrow(i, _):
        aloc[pl.ds(i * L, L)] = zeros
        return 0
    lax.fori_loop(0, (N * FPT) // L, zrow, 0)

    pltpu.sync_copy(hpre_hbm.at[wid], hloc)

    def piece(p, _):
        pltpu.sync_copy(src_hbm.at[pl.ds(p * PB, PB)], sidx)
        pltpu.sync_copy(dst_hbm.at[pl.ds(p * PB, PB)], didx)

        @plsc.parallel_loop(0, PB // L, unroll=4)
        def step(v):
            s16 = sidx[pl.ds(v * L, L)] * FPT
            d16 = didx[pl.ds(v * L, L)] * FPT
            for col in range(FPT):
                g = plsc.load_gather(hloc, [s16 + col])
                plsc.addupdate_scatter(aloc, [d16 + col], g)
        return 0
    lax.fori_loop(0, NP, piece, 0)

    pltpu.sync_copy(aloc, out_hbm.at[wid])


_agg_call = functools.partial(
    pl.kernel,
    out_type=jax.ShapeDtypeStruct((NW, N * FPT), jnp.float32),
    mesh=_MESH,
    compiler_params=pltpu.CompilerParams(needs_layout_passes=False),
    scratch_types=[
        pltpu.VMEM((PB,), jnp.int32),
        pltpu.VMEM((PB,), jnp.int32),
        pltpu.VMEM((N * FPT,), jnp.float32),
        pltpu.VMEM((N * FPT,), jnp.float32),
    ],
)(_agg_body)


def _to_slices(h):
    return h.reshape(N, NW, FPT).transpose(1, 0, 2).reshape(NW, N * FPT)


def _from_slices(a):
    return a.reshape(NW, N, FPT).transpose(1, 0, 2).reshape(N, D)


def _pre_body(degp_ref, x_ref, w_ref, dinv_ref, hpre_ref):
    deg = jnp.sum(degp_ref[...], axis=1, keepdims=True) + 1.0
    dinv = lax.rsqrt(jnp.maximum(deg, 1e-12))
    h = jnp.dot(x_ref[...], w_ref[...], preferred_element_type=jnp.float32)
    dinv_ref[...] = dinv
    hpre_ref[...] = h * dinv


def _pre_call(degp, x, w):
    return pl.pallas_call(
        _pre_body,
        out_shape=(
            jax.ShapeDtypeStruct((N, 1), jnp.float32),
            jax.ShapeDtypeStruct((N, D), jnp.float32),
        ),
    )(degp, x, w)


def _bn(t, g, be):
    mu = jnp.mean(t, axis=0, keepdims=True)
    var = jnp.mean((t - mu) * (t - mu), axis=0, keepdims=True)
    return (t - mu) * lax.rsqrt(var + 1e-5) * g + be


def _mid_body(agg_ref, hprev_ref, dinv_ref, g_ref, be_ref, w_ref, hnext_ref):
    dinv = dinv_ref[...]
    t = (agg_ref[...] + hprev_ref[...]) * dinv
    z = jnp.maximum(_bn(t, g_ref[...], be_ref[...]), 0.0)
    h = jnp.dot(z, w_ref[...], preferred_element_type=jnp.float32)
    hnext_ref[...] = h * dinv


def _mid_call(agg, hprev, dinv, g, be, w):
    return pl.pallas_call(
        _mid_body,
        out_shape=jax.ShapeDtypeStruct((N, D), jnp.float32),
    )(agg, hprev, dinv, g.reshape(1, D), be.reshape(1, D), w)


def _out_body(agg_ref, hprev_ref, dinv_ref, g_ref, be_ref, wc_ref, bc_ref,
              out_ref, emb_ref):
    t = (agg_ref[...] + hprev_ref[...]) * dinv_ref[...]
    emb = _bn(t, g_ref[...], be_ref[...])
    emb_ref[...] = emb
    out_ref[...] = jnp.dot(emb, wc_ref[...],
                           preferred_element_type=jnp.float32) + bc_ref[...]


def _out_call(agg, hprev, dinv, g, be, wc, bc):
    nout = wc.shape[1]
    return pl.pallas_call(
        _out_body,
        out_shape=(
            jax.ShapeDtypeStruct((N, nout), jnp.float32),
            jax.ShapeDtypeStruct((N, D), jnp.float32),
        ),
    )(agg, hprev, dinv, g.reshape(1, D), be.reshape(1, D), wc,
      bc.reshape(1, nout))


def kernel(x, edge_index, W1, b1, g1, be1, W2, b2, g2, be2, W3, b3, g3, be3,
           Wc, bc):
    src = edge_index[0]
    dst = edge_index[1]

    degp = _deg_call(dst)
    dinv, hpre1 = _pre_call(degp.T, x, W1)

    agg1 = _from_slices(_agg_call(_to_slices(hpre1), src, dst))
    hpre2 = _mid_call(agg1, hpre1, dinv, g1, be1, W2)

    agg2 = _from_slices(_agg_call(_to_slices(hpre2), src, dst))
    hpre3 = _mid_call(agg2, hpre2, dinv, g2, be2, W3)

    agg3 = _from_slices(_agg_call(_to_slices(hpre3), src, dst))
    out, emb = _out_call(agg3, hpre3, dinv, g3, be3, Wc, bc)
    return (out, emb)
